# baseline (speedup 1.0000x reference)
import functools

import jax
import jax.numpy as jnp
from jax import lax
from jax.experimental import pallas as pl
from jax.experimental.pallas import tpu as pltpu
from jax.experimental.pallas import tpu_sc as plsc

_NC = 2
_NS = 16
_NW = _NC * _NS
_K = 4


@functools.lru_cache(maxsize=None)
def _build_gather(nb, s, d):
    pairs_per_w = nb // _NW // 2
    s2 = 2 * s
    mesh = plsc.VectorSubcoreMesh(core_axis_name="c", subcore_axis_name="s")

    @functools.partial(
        pl.kernel,
        out_type=jax.ShapeDtypeStruct((nb, s, d), jnp.float32),
        mesh=mesh,
        scratch_types=[
            pltpu.VMEM((pairs_per_w, s2), jnp.int32),
            pltpu.VMEM((2, s2, d), jnp.float32),
            pltpu.SemaphoreType.DMA((2,)),
            pltpu.SemaphoreType.DMA((2,)),
        ],
    )
    def gather_kernel(table_hbm, idx_hbm, out_hbm, idx_v, rows_v, gsem, ssem):
        wid = lax.axis_index("s") * _NC + lax.axis_index("c")
        base = wid * pairs_per_w
        pltpu.sync_copy(idx_hbm.at[pl.ds(base, pairs_per_w)], idx_v)
        pltpu.async_copy(table_hbm.at[idx_v.at[0]], rows_v.at[0], gsem.at[0])

        @pl.loop(0, pairs_per_w, step=2)
        def round_(r):
            for sub in range(2):
                c = r + sub
                slot = sub
                other = 1 - sub
                pltpu.make_async_copy(
                    table_hbm.at[idx_v.at[c]], rows_v.at[slot], gsem.at[slot]
                ).wait()

                @pl.when(c + 1 < pairs_per_w)
                def _():
                    @pl.when(c >= 1)
                    def _():
                        for h in range(2):
                            pltpu.make_async_copy(
                                rows_v.at[other].at[pl.ds(h * s, s)],
                                out_hbm.at[base],
                                ssem.at[other],
                            ).wait()

                    pltpu.async_copy(
                        table_hbm.at[idx_v.at[c + 1]], rows_v.at[other], gsem.at[other]
                    )

                for h in range(2):
                    pltpu.async_copy(
                        rows_v.at[slot].at[pl.ds(h * s, s)],
                        out_hbm.at[2 * (base + c) + h],
                        ssem.at[slot],
                    )

        for slot in range(2):
            for h in range(2):
                pltpu.make_async_copy(
                    rows_v.at[slot].at[pl.ds(h * s, s)],
                    out_hbm.at[base],
                    ssem.at[slot],
                ).wait()

    return gather_kernel


def kernel(words_ids, table):
    b, s = words_ids.shape
    v, d = table.shape
    nb = b // _K
    idx = words_ids.reshape(_K, nb // 2, 2 * s).astype(jnp.int32)
    fn = _build_gather(nb, s, d)
    outs = [fn(table, idx[k]) for k in range(_K)]
    return jnp.concatenate(outs, axis=0)


# speedup vs baseline: 1.1039x; 1.1039x over previous
import functools

import jax
import jax.numpy as jnp
from jax import lax
from jax.experimental import pallas as pl
from jax.experimental.pallas import tpu as pltpu
from jax.experimental.pallas import tpu_sc as plsc

_NC = 2
_NS = 16
_NW = _NC * _NS
_K = 4


@functools.lru_cache(maxsize=None)
def _build_gather(nb, s, d):
    pairs_per_w = nb // _NW // 2
    s2 = 2 * s
    mesh = plsc.VectorSubcoreMesh(core_axis_name="c", subcore_axis_name="s")

    @functools.partial(
        pl.kernel,
        out_type=jax.ShapeDtypeStruct((nb, s, d), jnp.float32),
        mesh=mesh,
        scratch_types=[
            pltpu.VMEM((pairs_per_w, s2), jnp.int32),
            pltpu.VMEM((2, s2, d), jnp.float32),
            pltpu.SemaphoreType.DMA((2,)),
            pltpu.SemaphoreType.DMA((2,)),
        ],
    )
    def gather_kernel(table_hbm, idx_hbm, out_hbm, idx_v, rows_v, gsem, ssem):
        wid = lax.axis_index("s") * _NC + lax.axis_index("c")
        base = wid * pairs_per_w
        pltpu.sync_copy(idx_hbm.at[pl.ds(base, pairs_per_w)], idx_v)
        pltpu.async_copy(table_hbm.at[idx_v.at[0]], rows_v.at[0], gsem.at[0])

        @pl.loop(0, pairs_per_w, step=2)
        def round_(r):
            for sub in range(2):
                c = r + sub
                slot = sub
                other = 1 - sub
                pltpu.make_async_copy(
                    table_hbm.at[idx_v.at[c]], rows_v.at[slot], gsem.at[slot]
                ).wait()

                @pl.when(c + 1 < pairs_per_w)
                def _():
                    @pl.when(c >= 1)
                    def _():
                        for h in range(2):
                            pltpu.make_async_copy(
                                rows_v.at[other].at[pl.ds(h * s, s)],
                                out_hbm.at[base],
                                ssem.at[other],
                            ).wait()

                    pltpu.async_copy(
                        table_hbm.at[idx_v.at[c + 1]], rows_v.at[other], gsem.at[other]
                    )

                for h in range(2):
                    pltpu.async_copy(
                        rows_v.at[slot].at[pl.ds(h * s, s)],
                        out_hbm.at[2 * (base + c) + h],
                        ssem.at[slot],
                    )

        for slot in range(2):
            for h in range(2):
                pltpu.make_async_copy(
                    rows_v.at[slot].at[pl.ds(h * s, s)],
                    out_hbm.at[base],
                    ssem.at[slot],
                ).wait()

    return gather_kernel


def kernel(words_ids, table):
    b, s = words_ids.shape
    v, d = table.shape
    nb = b // _K
    idx = words_ids.reshape(_K, nb // 2, 2 * s).astype(jnp.int32)
    fn = _build_gather(nb, s, d)
    out = jnp.zeros((b, s, d), jnp.float32)
    for k in range(_K):
        out = lax.dynamic_update_slice(out, fn(table, idx[k]), (k * nb, 0, 0))
    return out
